# baseline (speedup 1.0000x reference)
import functools

import jax
import jax.numpy as jnp
from jax import lax
from jax.experimental import pallas as pl
from jax.experimental.pallas import tpu as pltpu
from jax.experimental.pallas import tpu_sc as plsc

N = 10000
E = 320000
D_IN = 128
D_H = 128
D_OUT = 40

NC = 2
NS = 16
LANES = 16

N_PAD = 10240
STRIPE = N_PAD // NS
E_PAD = 327680
EPS = E_PAD // NS
ZR = 64


def _spmm_kernel(hh):
  eps2 = E_PAD // (NC * NS)
  k = 80 if hh == 128 else 256
  nchunk = eps2 // k
  zr = 32
  mesh = plsc.VectorSubcoreMesh(core_axis_name="c", subcore_axis_name="s",
                                num_cores=NC, num_subcores=NS)

  @functools.partial(
      pl.kernel,
      mesh=mesh,
      out_type=jax.ShapeDtypeStruct((NC, N_PAD, hh), jnp.float32),
      scratch_types=[
          pltpu.VMEM_SHARED((N_PAD, hh), jnp.float32),
          pltpu.VMEM((eps2,), jnp.int32),
          pltpu.VMEM((eps2,), jnp.int32),
          pltpu.VMEM((k, hh), jnp.float32),
          pltpu.VMEM((k, hh), jnp.float32),
          pltpu.VMEM((zr, hh), jnp.float32),
          pltpu.SemaphoreType.DMA,
          pltpu.SemaphoreType.DMA,
      ],
      compiler_params=pltpu.CompilerParams(use_tc_tiling_on_sc=False),
  )
  def spmm(y_hbm, idx_hbm, out_hbm, sh_agg, srcall, dstall, rows0, rows1,
           zbuf, gsem0, gsem1):
    c = lax.axis_index("c")
    s = lax.axis_index("s")

    zv = jnp.zeros((LANES,), jnp.float32)
    per_row = hh // LANES

    @pl.loop(0, zr * per_row)
    def _(i):
      r = i // per_row
      col = (i % per_row) * LANES
      zbuf[r, pl.ds(col, LANES)] = zv

    @pl.loop(0, STRIPE // zr)
    def _(t):
      pltpu.sync_copy(zbuf, sh_agg.at[pl.ds(s * STRIPE + t * zr, zr)])

    e0 = (c * NS + s) * eps2
    pltpu.sync_copy(idx_hbm.at[0, pl.ds(e0, eps2)], srcall)
    pltpu.sync_copy(idx_hbm.at[1, pl.ds(e0, eps2)], dstall)
    plsc.subcore_barrier()

    def gather(t, rows, sem):
      return pltpu.async_copy(
          y_hbm.at[srcall.at[pl.ds(t * k, k)]], rows, sem)

    def scatter(t, rows):
      pltpu.sync_copy(rows, sh_agg.at[dstall.at[pl.ds(t * k, k)]], add=True)

    gather(0, rows0, gsem0)

    @pl.loop(0, nchunk // 2)
    def _(u):
      t = u * 2
      g1 = gather(t + 1, rows1, gsem1)
      pltpu.make_async_copy(
          y_hbm.at[srcall.at[pl.ds(t * k, k)]], rows0, gsem0).wait()
      scatter(t, rows0)

      @pl.when(t + 2 < nchunk)
      def _():
        gather(t + 2, rows0, gsem0)

      g1.wait()
      scatter(t + 1, rows1)

    plsc.subcore_barrier()
    pltpu.sync_copy(sh_agg.at[pl.ds(s * STRIPE, STRIPE)],
                    out_hbm.at[c, pl.ds(s * STRIPE, STRIPE)])

  return spmm


def _bincount_kernel():
  mesh = plsc.VectorSubcoreMesh(core_axis_name="c", subcore_axis_name="s",
                                num_cores=NC, num_subcores=NS)

  @functools.partial(
      pl.kernel,
      mesh=mesh,
      out_type=jax.ShapeDtypeStruct((NC * NS * N_PAD,), jnp.int32),
      scratch_types=[
          pltpu.VMEM((N_PAD,), jnp.int32),
          pltpu.VMEM((EPS,), jnp.int32),
      ],
      compiler_params=pltpu.CompilerParams(use_tc_tiling_on_sc=False,
                                           needs_layout_passes=False),
  )
  def bincount(idx_hbm, out_hbm, cnt_v, idx_v):
    c = lax.axis_index("c")
    s = lax.axis_index("s")
    zv = jnp.zeros((LANES,), jnp.int32)

    @pl.loop(0, N_PAD // LANES)
    def _(i):
      cnt_v[pl.ds(i * LANES, LANES)] = zv

    pltpu.sync_copy(idx_hbm.at[c, pl.ds(s * EPS, EPS)], idx_v)
    ones = jnp.ones((LANES,), jnp.int32)

    @pl.loop(0, EPS // LANES)
    def _(i):
      v = idx_v[pl.ds(i * LANES, LANES)]
      plsc.addupdate_scatter(cnt_v, [v], ones)

    pltpu.sync_copy(cnt_v, out_hbm.at[pl.ds((c * NS + s) * N_PAD, N_PAD)])

  return bincount


ROWB = 256
GRID = N_PAD // ROWB


def _norm_body(deg_ref, norm_ref):
  d = jnp.sum(deg_ref[...], axis=1).astype(jnp.float32)
  norm_ref[...] = jnp.where(d > 0.0, lax.rsqrt(jnp.maximum(d, 1.0)), 0.0)


def _norms(deg_partials):
  return pl.pallas_call(
      _norm_body,
      out_shape=jax.ShapeDtypeStruct(
          (NC, N_PAD // 128, 128), jnp.float32),
  )(deg_partials)


def _mm_first_body(x_ref, onorm_ref, w_ref, o_ref):
  o_ref[...] = jnp.dot((x_ref[...] * onorm_ref[...]), w_ref[...],
                       preferred_element_type=jnp.float32,
                       precision=lax.Precision.HIGHEST)


def _mm_first(x, onorm, w):
  dh = w.shape[1]
  return pl.pallas_call(
      _mm_first_body,
      grid=(GRID,),
      in_specs=[
          pl.BlockSpec((ROWB, D_IN), lambda i: (i, 0)),
          pl.BlockSpec((ROWB, 1), lambda i: (i, 0)),
          pl.BlockSpec((D_IN, dh), lambda i: (0, 0)),
      ],
      out_specs=pl.BlockSpec((ROWB, dh), lambda i: (i, 0)),
      out_shape=jax.ShapeDtypeStruct((N_PAD, dh), jnp.float32),
  )(x, onorm, w)


def _mm_mid_body(a_ref, inorm_ref, onorm_ref, b_ref, w_ref, o_ref):
  a = a_ref[0] + a_ref[1]
  h = jnp.maximum(a * inorm_ref[...] + b_ref[...], 0.0)
  o_ref[...] = jnp.dot(h * onorm_ref[...], w_ref[...],
                       preferred_element_type=jnp.float32,
                       precision=lax.Precision.HIGHEST)


def _mm_mid(agg, inorm, onorm, b, w):
  dh = w.shape[1]
  return pl.pallas_call(
      _mm_mid_body,
      grid=(GRID,),
      in_specs=[
          pl.BlockSpec((NC, ROWB, D_H), lambda i: (0, i, 0)),
          pl.BlockSpec((ROWB, 1), lambda i: (i, 0)),
          pl.BlockSpec((ROWB, 1), lambda i: (i, 0)),
          pl.BlockSpec((1, D_H), lambda i: (0, 0)),
          pl.BlockSpec((D_H, dh), lambda i: (0, 0)),
      ],
      out_specs=pl.BlockSpec((ROWB, dh), lambda i: (i, 0)),
      out_shape=jax.ShapeDtypeStruct((N_PAD, dh), jnp.float32),
  )(agg, inorm, onorm, b, w)


def _final_body(a_ref, inorm_ref, b_ref, o_ref):
  a = a_ref[0] + a_ref[1]
  o_ref[...] = a * inorm_ref[...] + b_ref[...]


def _final(agg, inorm, b_pad, dh):
  return pl.pallas_call(
      _final_body,
      grid=(GRID,),
      in_specs=[
          pl.BlockSpec((NC, ROWB, dh), lambda i: (0, i, 0)),
          pl.BlockSpec((ROWB, 1), lambda i: (i, 0)),
          pl.BlockSpec((1, dh), lambda i: (0, 0)),
      ],
      out_specs=pl.BlockSpec((ROWB, dh), lambda i: (i, 0)),
      out_shape=jax.ShapeDtypeStruct((N_PAD, dh), jnp.float32),
  )(agg, inorm, b_pad)


def kernel(features, edge_index, W1, b1, W2, b2, W3, b3):
  x = jnp.pad(features, ((0, N_PAD - N), (0, 0)))
  pad_e = jnp.full((2, E_PAD - E), N_PAD - 1, dtype=jnp.int32)
  idx = jnp.concatenate([edge_index, pad_e], axis=1)
  w3p = jnp.pad(W3, ((0, 0), (0, 64 - D_OUT)))
  b3p = jnp.pad(b3, (0, 64 - D_OUT)).reshape(1, 64)

  deg = _bincount_kernel()(idx).reshape(NC, NS, N_PAD // 128, 128)
  norms = _norms(deg)
  onorm = norms[0].reshape(N_PAD, 1)
  inorm = norms[1].reshape(N_PAD, 1)

  spmm128 = _spmm_kernel(D_H)
  spmm64 = _spmm_kernel(64)

  y1 = _mm_first(x, onorm, W1)
  a1 = spmm128(y1, idx)
  y2 = _mm_mid(a1, inorm, onorm, b1.reshape(1, D_H), W2)
  a2 = spmm128(y2, idx)
  y3 = _mm_mid(a2, inorm, onorm, b2.reshape(1, D_H), w3p)
  a3 = spmm64(y3, idx)
  out = _final(a3, inorm, b3p, 64)
  return out[:N, :D_OUT]


# speedup vs baseline: 1.6482x; 1.6482x over previous
import functools

import jax
import jax.numpy as jnp
from jax import lax
from jax.experimental import pallas as pl
from jax.experimental.pallas import tpu as pltpu
from jax.experimental.pallas import tpu_sc as plsc

N = 10000
E = 320000
D_IN = 128
D_H = 128
D_OUT = 40

NC = 2
NS = 16
LANES = 16

N_PAD = 10240
STRIPE = N_PAD // NS
E_PAD = 327680
EPS = E_PAD // NS
ZR = 32


def _spmm_kernel(hh):
  k = 512 if hh == 64 else 1024
  nchunk = EPS // k
  mesh = plsc.VectorSubcoreMesh(core_axis_name="c", subcore_axis_name="s",
                                num_cores=NC, num_subcores=NS)

  @functools.partial(
      pl.kernel,
      mesh=mesh,
      out_type=jax.ShapeDtypeStruct((NC, N_PAD, hh), jnp.float32),
      scratch_types=[
          pltpu.VMEM_SHARED((N_PAD, hh), jnp.float32),
          pltpu.VMEM((EPS,), jnp.int32),
          pltpu.VMEM((k,), jnp.int32),
          pltpu.VMEM((k,), jnp.int32),
          pltpu.VMEM((k, hh), jnp.float32),
          pltpu.VMEM((k, hh), jnp.float32),
          pltpu.VMEM((ZR, hh), jnp.float32),
          pltpu.SemaphoreType.DMA,
          pltpu.SemaphoreType.DMA,
          pltpu.SemaphoreType.DMA,
          pltpu.SemaphoreType.DMA,
      ],
      compiler_params=pltpu.CompilerParams(use_tc_tiling_on_sc=False),
  )
  def spmm(y_hbm, idx_hbm, out_hbm, sh_agg, srcall, dst0, dst1, rows0, rows1,
           zbuf, gsem0, gsem1, dsem0, dsem1):
    c = lax.axis_index("c")
    s = lax.axis_index("s")

    zv = jnp.zeros((LANES,), jnp.float32)
    per_row = hh // LANES

    @pl.loop(0, ZR * per_row)
    def _(i):
      r = i // per_row
      col = (i % per_row) * LANES
      zbuf[r, pl.ds(col, LANES)] = zv

    @pl.loop(0, STRIPE // ZR)
    def _(t):
      pltpu.sync_copy(zbuf, sh_agg.at[pl.ds(s * STRIPE + t * ZR, ZR)])

    pltpu.sync_copy(idx_hbm.at[0, pl.ds(s * EPS, EPS)], srcall)
    plsc.subcore_barrier()

    def gather(t, rows, sem):
      return pltpu.async_copy(
          y_hbm.at[c].at[srcall.at[pl.ds(t * k, k)]], rows, sem)

    def dst_load(t, buf, sem):
      return pltpu.async_copy(
          idx_hbm.at[1, pl.ds(s * EPS + t * k, k)], buf, sem)

    def scatter(rows, buf):
      pltpu.sync_copy(rows, sh_agg.at[buf], add=True)

    dst_load(0, dst0, dsem0)
    gather(0, rows0, gsem0)
    dst_load(1, dst1, dsem1)

    @pl.loop(0, nchunk // 2)
    def _(u):
      t = u * 2
      g1 = gather(t + 1, rows1, gsem1)
      pltpu.make_async_copy(
          y_hbm.at[c].at[srcall.at[pl.ds(t * k, k)]], rows0, gsem0).wait()
      pltpu.make_async_copy(
          idx_hbm.at[1, pl.ds(s * EPS + t * k, k)], dst0, dsem0).wait()
      scatter(rows0, dst0)

      @pl.when(t + 2 < nchunk)
      def _():
        dst_load(t + 2, dst0, dsem0)
        gather(t + 2, rows0, gsem0)

      g1.wait()
      pltpu.make_async_copy(
          idx_hbm.at[1, pl.ds(s * EPS + (t + 1) * k, k)], dst1, dsem1).wait()
      scatter(rows1, dst1)

      @pl.when(t + 3 < nchunk)
      def _():
        dst_load(t + 3, dst1, dsem1)

    plsc.subcore_barrier()
    pltpu.sync_copy(sh_agg.at[pl.ds(s * STRIPE, STRIPE)],
                    out_hbm.at[c, pl.ds(s * STRIPE, STRIPE)])

  return spmm


def _bincount_kernel():
  mesh = plsc.VectorSubcoreMesh(core_axis_name="c", subcore_axis_name="s",
                                num_cores=NC, num_subcores=NS)

  @functools.partial(
      pl.kernel,
      mesh=mesh,
      out_type=jax.ShapeDtypeStruct((NC * NS * N_PAD,), jnp.int32),
      scratch_types=[
          pltpu.VMEM((N_PAD,), jnp.int32),
          pltpu.VMEM((EPS,), jnp.int32),
      ],
      compiler_params=pltpu.CompilerParams(use_tc_tiling_on_sc=False,
                                           needs_layout_passes=False),
  )
  def bincount(idx_hbm, out_hbm, cnt_v, idx_v):
    c = lax.axis_index("c")
    s = lax.axis_index("s")
    zv = jnp.zeros((LANES,), jnp.int32)

    @pl.loop(0, N_PAD // LANES)
    def _(i):
      cnt_v[pl.ds(i * LANES, LANES)] = zv

    pltpu.sync_copy(idx_hbm.at[c, pl.ds(s * EPS, EPS)], idx_v)
    ones = jnp.ones((LANES,), jnp.int32)

    @pl.loop(0, EPS // LANES)
    def _(i):
      v = idx_v[pl.ds(i * LANES, LANES)]
      plsc.addupdate_scatter(cnt_v, [v], ones)

    pltpu.sync_copy(cnt_v, out_hbm.at[pl.ds((c * NS + s) * N_PAD, N_PAD)])

  return bincount


ROWB = 256
GRID = N_PAD // ROWB


def _norm_body(deg_ref, norm_ref):
  d = jnp.sum(deg_ref[...], axis=1).astype(jnp.float32)
  norm_ref[...] = jnp.where(d > 0.0, lax.rsqrt(jnp.maximum(d, 1.0)), 0.0)


def _norms(deg_partials):
  return pl.pallas_call(
      _norm_body,
      out_shape=jax.ShapeDtypeStruct(
          (NC, N_PAD // 128, 128), jnp.float32),
  )(deg_partials)


def _mm_first_body(x_ref, onorm_ref, w_ref, o_ref):
  y = jnp.dot((x_ref[...] * onorm_ref[...]), w_ref[...],
              preferred_element_type=jnp.float32,
              precision=lax.Precision.HIGHEST)
  h = y.shape[-1] // 2
  o_ref[0] = y[:, :h]
  o_ref[1] = y[:, h:]


def _mm_first(x, onorm, w):
  dh = w.shape[1]
  return pl.pallas_call(
      _mm_first_body,
      grid=(GRID,),
      in_specs=[
          pl.BlockSpec((ROWB, D_IN), lambda i: (i, 0)),
          pl.BlockSpec((ROWB, 1), lambda i: (i, 0)),
          pl.BlockSpec((D_IN, dh), lambda i: (0, 0)),
      ],
      out_specs=pl.BlockSpec((NC, ROWB, dh // 2), lambda i: (0, i, 0)),
      out_shape=jax.ShapeDtypeStruct((NC, N_PAD, dh // 2), jnp.float32),
  )(x, onorm, w)


def _mm_mid_body(a_ref, inorm_ref, onorm_ref, b_ref, w_ref, o_ref):
  a = jnp.concatenate([a_ref[0], a_ref[1]], axis=1)
  h = jnp.maximum(a * inorm_ref[...] + b_ref[...], 0.0)
  y = jnp.dot(h * onorm_ref[...], w_ref[...],
              preferred_element_type=jnp.float32,
              precision=lax.Precision.HIGHEST)
  hh = y.shape[-1] // 2
  o_ref[0] = y[:, :hh]
  o_ref[1] = y[:, hh:]


def _mm_mid(agg, inorm, onorm, b, w):
  dh = w.shape[1]
  return pl.pallas_call(
      _mm_mid_body,
      grid=(GRID,),
      in_specs=[
          pl.BlockSpec((NC, ROWB, D_H // 2), lambda i: (0, i, 0)),
          pl.BlockSpec((ROWB, 1), lambda i: (i, 0)),
          pl.BlockSpec((ROWB, 1), lambda i: (i, 0)),
          pl.BlockSpec((1, D_H), lambda i: (0, 0)),
          pl.BlockSpec((D_H, dh), lambda i: (0, 0)),
      ],
      out_specs=pl.BlockSpec((NC, ROWB, dh // 2), lambda i: (0, i, 0)),
      out_shape=jax.ShapeDtypeStruct((NC, N_PAD, dh // 2), jnp.float32),
  )(agg, inorm, onorm, b, w)


def _final_body(a_ref, inorm_ref, b_ref, o_ref):
  a = jnp.concatenate([a_ref[0], a_ref[1]], axis=1)
  o_ref[...] = a * inorm_ref[...] + b_ref[...]


def _final(agg, inorm, b_pad, dh):
  return pl.pallas_call(
      _final_body,
      grid=(GRID,),
      in_specs=[
          pl.BlockSpec((NC, ROWB, dh // 2), lambda i: (0, i, 0)),
          pl.BlockSpec((ROWB, 1), lambda i: (i, 0)),
          pl.BlockSpec((1, dh), lambda i: (0, 0)),
      ],
      out_specs=pl.BlockSpec((ROWB, dh), lambda i: (i, 0)),
      out_shape=jax.ShapeDtypeStruct((N_PAD, dh), jnp.float32),
  )(agg, inorm, b_pad)


def kernel(features, edge_index, W1, b1, W2, b2, W3, b3):
  x = jnp.pad(features, ((0, N_PAD - N), (0, 0)))
  pad_e = jnp.full((2, E_PAD - E), N_PAD - 1, dtype=jnp.int32)
  idx = jnp.concatenate([edge_index, pad_e], axis=1)
  w3p = jnp.pad(W3, ((0, 0), (0, 64 - D_OUT)))
  b3p = jnp.pad(b3, (0, 64 - D_OUT)).reshape(1, 64)

  deg = _bincount_kernel()(idx).reshape(NC, NS, N_PAD // 128, 128)
  norms = _norms(deg)
  onorm = norms[0].reshape(N_PAD, 1)
  inorm = norms[1].reshape(N_PAD, 1)

  spmm128 = _spmm_kernel(D_H // 2)
  spmm64 = _spmm_kernel(32)

  y1 = _mm_first(x, onorm, W1)
  a1 = spmm128(y1, idx)
  y2 = _mm_mid(a1, inorm, onorm, b1.reshape(1, D_H), W2)
  a2 = spmm128(y2, idx)
  y3 = _mm_mid(a2, inorm, onorm, b2.reshape(1, D_H), w3p)
  a3 = spmm64(y3, idx)
  out = _final(a3, inorm, b3p, 64)
  return out[:N, :D_OUT]
